# baseline (speedup 1.0000x reference)
import functools

import jax
import jax.numpy as jnp
from jax import lax
from jax.experimental import pallas as pl
from jax.experimental.pallas import tpu as pltpu
from jax.experimental.pallas import tpu_sc as plsc

N = 10000
E = 320000
C = 128
NW = 32
NCH = E // C
RPS = 624


def _make_spmm(D, NBUF, H):
    mesh = plsc.VectorSubcoreMesh(core_axis_name="c", subcore_axis_name="s")
    KV = D // 16
    GP = NBUF - 2
    CE = H * C
    nchd = E // CE
    realmax = (nchd + NW - 1) // NW
    nchmax = ((realmax + NBUF - 1) // NBUF) * NBUF

    per_set = [
        pltpu.VMEM((3 * H, C), jnp.int32),
        pltpu.VMEM((H, C), jnp.int32),
        pltpu.VMEM((CE, D), jnp.float32),
        pltpu.SemaphoreType.DMA,
        pltpu.SemaphoreType.DMA,
        pltpu.SemaphoreType.DMA,
    ]

    @functools.partial(
        pl.kernel,
        out_type=jax.ShapeDtypeStruct((2 * N, D), jnp.float32),
        mesh=mesh,
        compiler_params=pltpu.CompilerParams(
            needs_layout_passes=False, use_tc_tiling_on_sc=False),
        scratch_types=per_set * NBUF
        + [pltpu.VMEM_SHARED((N, D), jnp.float32)],
    )
    def spmm(x_hbm, idxw_hbm, out_hbm, *scratch):
        sets = tuple(tuple(scratch[6 * i:6 * i + 6]) for i in range(NBUF))
        acc = scratch[6 * NBUF]
        c = lax.axis_index("c")
        s = lax.axis_index("s")
        wid = s * 2 + c
        r0 = s * RPS
        nch = (nchd - wid + NW - 1) // NW

        def start_idx(t, st):
            ibuf, _, _, isem, _, _ = st
            pltpu.async_copy(idxw_hbm.at[wid + NW * t], ibuf, isem)

        def wait_idx(t, st):
            ibuf, _, _, isem, _, _ = st
            pltpu.make_async_copy(idxw_hbm.at[wid + NW * t], ibuf,
                                  isem).wait()

        def start_gather(st):
            ibuf, _, gbuf, _, gsem, _ = st
            for r in range(H):
                pltpu.async_copy(x_hbm.at[ibuf.at[r]],
                                 gbuf.at[pl.ds(r * C, C)], gsem)

        def wait_gather(st):
            _, _, gbuf, _, gsem, _ = st
            pltpu.make_async_copy(x_hbm.at[pl.ds(0, CE)], gbuf, gsem).wait()

        def start_scatter(st):
            _, ridx, gbuf, _, _, ssem = st
            for r in range(H):
                pltpu.async_copy(gbuf.at[pl.ds(r * C, C)],
                                 acc.at[ridx.at[r]], ssem, add=True)

        def wait_scatter(st):
            _, _, gbuf, _, _, ssem = st
            pltpu.make_async_copy(x_hbm.at[pl.ds(0, CE)], gbuf, ssem).wait()

        zbuf = sets[NBUF - 1][2]

        def zrow(i, carry):
            for k in range(KV):
                zbuf[i, pl.ds(k * 16, 16)] = jnp.zeros((16,), jnp.float32)
            return carry

        lax.fori_loop(0, CE, zrow, 0, unroll=4)
        for q in range(RPS // CE):
            pltpu.sync_copy(zbuf, acc.at[pl.ds(r0 + q * CE, CE)])
        rem = RPS - (RPS // CE) * CE
        pltpu.sync_copy(zbuf.at[pl.ds(0, rem)],
                        acc.at[pl.ds(r0 + (RPS // CE) * CE, rem)])

        @pl.when(s == 15)
        def _zero_tail():
            pltpu.sync_copy(zbuf.at[pl.ds(0, N - 16 * RPS)],
                            acc.at[pl.ds(16 * RPS, N - 16 * RPS)])

        plsc.subcore_barrier()

        for i in range(NBUF):
            start_idx(i, sets[i])
        for i in range(GP):
            wait_idx(i, sets[i])
            start_gather(sets[i])

        def step(t, i):
            cur = sets[i]
            nxg = sets[(i + GP) % NBUF]
            ibuf, ridx, gbuf, _, _, _ = cur

            @pl.when((t >= 2) & (t - 2 < nch))
            def _free_next_gbuf():
                wait_scatter(nxg)

            @pl.when(t + GP < nch)
            def _prefetch_gather():
                wait_idx(t + GP, nxg)
                start_gather(nxg)

            @pl.when(t < nch)
            def _process():
                wait_gather(cur)

                def edge_body(e, carry):
                    for r in range(H):
                        bw = plsc.bitcast(
                            plsc.load_gather(
                                ibuf,
                                [jnp.full((16,), 2 * H + r, jnp.int32),
                                 jnp.full((16,), e, jnp.int32)]),
                            jnp.float32)
                        for k in range(KV):
                            sl = pl.ds(k * 16, 16)
                            gbuf[r * C + e, sl] = gbuf[r * C + e, sl] * bw
                    return carry

                lax.fori_loop(0, C, edge_body, 0, unroll=4)
                for r in range(H):
                    for k in range(8):
                        sl = pl.ds(k * 16, 16)
                        ridx[r, sl] = ibuf[H + r, sl]
                start_scatter(cur)

            @pl.when(t + NBUF < nch)
            def _prefetch_idx():
                start_idx(t + NBUF, cur)

        def ring_body(u, carry):
            for i in range(NBUF):
                step(NBUF * u + i, i)
            return carry

        lax.fori_loop(0, nchmax // NBUF, ring_body, 0)

        for t in range(nchmax - 2, realmax):
            _st = sets[t % NBUF]

            @pl.when(t < nch)
            def _drain(_st=_st):
                wait_scatter(_st)

        plsc.subcore_barrier()
        pltpu.sync_copy(acc.at[pl.ds(r0, RPS)],
                        out_hbm.at[pl.ds(c * N + r0, RPS)])

        @pl.when(s == 15)
        def _write_tail():
            pltpu.sync_copy(acc.at[pl.ds(16 * RPS, N - 16 * RPS)],
                            out_hbm.at[pl.ds(c * N + 16 * RPS, N - 16 * RPS)])

    return spmm


def _make_dense(Din, Dout, R):
    def body(p_ref, w_ref, b_ref, o_ref):
        x = p_ref[0] + p_ref[1]
        y = jnp.dot(x, w_ref[...], preferred_element_type=jnp.float32,
                    precision=lax.Precision.HIGHEST)
        y = y + b_ref[...]
        nrm = jnp.sqrt(jnp.sum(y * y, axis=1, keepdims=True))
        o_ref[...] = y / jnp.maximum(nrm, 1e-12)

    return pl.pallas_call(
        body,
        grid=(N // R,),
        in_specs=[
            pl.BlockSpec((2, R, Din), lambda i: (0, i, 0)),
            pl.BlockSpec((Din, Dout), lambda i: (0, 0)),
            pl.BlockSpec((1, Dout), lambda i: (0, 0)),
        ],
        out_specs=pl.BlockSpec((R, Dout), lambda i: (i, 0)),
        out_shape=jax.ShapeDtypeStruct((N, Dout), jnp.float32),
    )


_spmm_128 = _make_spmm(128, 3, 1)
_spmm_64 = _make_spmm(64, 4, 2)
_dense_0 = _make_dense(128, 64, 1000)
_dense_1 = _make_dense(64, 128, 1000)


def _pack_idxw(edge_index, edge_weight, H):
    nchd = E // (H * C)
    return jnp.concatenate(
        [edge_index[1].reshape(nchd, H, C),
         edge_index[0].reshape(nchd, H, C),
         lax.bitcast_convert_type(edge_weight, jnp.int32).reshape(
             nchd, H, C)],
        axis=1)


def kernel(fts, edge_index, edge_weight, W_gc_0, b_gc_0, W_gc_1, b_gc_1):
    idxw1 = _pack_idxw(edge_index, edge_weight, 1)
    idxw2 = _pack_idxw(edge_index, edge_weight, 2)
    p0 = _spmm_128(fts, idxw1).reshape(2, N, 128)
    ego = _dense_0(p0, W_gc_0, b_gc_0)
    p1 = _spmm_64(ego, idxw2).reshape(2, N, 64)
    return _dense_1(p1, W_gc_1, b_gc_1)


# speedup vs baseline: 1.0172x; 1.0172x over previous
import functools

import jax
import jax.numpy as jnp
from jax import lax
from jax.experimental import pallas as pl
from jax.experimental.pallas import tpu as pltpu
from jax.experimental.pallas import tpu_sc as plsc

N = 10000
E = 320000
C = 128
NW = 32
NCH = E // C
RPS = 624


def _make_spmm(D, NBUF, H):
    mesh = plsc.VectorSubcoreMesh(core_axis_name="c", subcore_axis_name="s")
    KV = D // 16
    GP = NBUF - 2
    CE = H * C
    nchd = E // CE
    realmax = (nchd + NW - 1) // NW
    nchmax = ((realmax + NBUF - 1) // NBUF) * NBUF

    per_set = [
        pltpu.VMEM((3 * H, C), jnp.int32),
        pltpu.VMEM((H, C), jnp.int32),
        pltpu.VMEM((CE, D), jnp.float32),
        pltpu.SemaphoreType.DMA,
        pltpu.SemaphoreType.DMA,
        pltpu.SemaphoreType.DMA,
    ]

    @functools.partial(
        pl.kernel,
        out_type=jax.ShapeDtypeStruct((2 * N, D), jnp.float32),
        mesh=mesh,
        compiler_params=pltpu.CompilerParams(
            needs_layout_passes=False, use_tc_tiling_on_sc=False),
        scratch_types=per_set * NBUF
        + [pltpu.VMEM_SHARED((N, D), jnp.float32)],
    )
    def spmm(x_hbm, idxw_hbm, out_hbm, *scratch):
        sets = tuple(tuple(scratch[6 * i:6 * i + 6]) for i in range(NBUF))
        acc = scratch[6 * NBUF]
        c = lax.axis_index("c")
        s = lax.axis_index("s")
        wid = s * 2 + c
        r0 = s * RPS
        nch = (nchd - wid + NW - 1) // NW

        def start_idx(t, st):
            ibuf, _, _, isem, _, _ = st
            pltpu.async_copy(idxw_hbm.at[wid + NW * t], ibuf, isem)

        def wait_idx(t, st):
            ibuf, _, _, isem, _, _ = st
            pltpu.make_async_copy(idxw_hbm.at[wid + NW * t], ibuf,
                                  isem).wait()

        def start_gather(st):
            ibuf, _, gbuf, _, gsem, _ = st
            for r in range(H):
                pltpu.async_copy(x_hbm.at[ibuf.at[r]],
                                 gbuf.at[pl.ds(r * C, C)], gsem)

        def wait_gather(st):
            _, _, gbuf, _, gsem, _ = st
            pltpu.make_async_copy(x_hbm.at[pl.ds(0, CE)], gbuf, gsem).wait()

        def start_scatter(st):
            _, ridx, gbuf, _, _, ssem = st
            for r in range(H):
                pltpu.async_copy(gbuf.at[pl.ds(r * C, C)],
                                 acc.at[ridx.at[r]], ssem, add=True)

        def wait_scatter(st):
            _, _, gbuf, _, _, ssem = st
            pltpu.make_async_copy(x_hbm.at[pl.ds(0, CE)], gbuf, ssem).wait()

        zbuf = sets[NBUF - 1][2]

        def zrow(i, carry):
            for k in range(KV):
                zbuf[i, pl.ds(k * 16, 16)] = jnp.zeros((16,), jnp.float32)
            return carry

        lax.fori_loop(0, CE, zrow, 0, unroll=4)
        for q in range(RPS // CE):
            pltpu.sync_copy(zbuf, acc.at[pl.ds(r0 + q * CE, CE)])
        rem = RPS - (RPS // CE) * CE
        pltpu.sync_copy(zbuf.at[pl.ds(0, rem)],
                        acc.at[pl.ds(r0 + (RPS // CE) * CE, rem)])

        @pl.when(s == 15)
        def _zero_tail():
            pltpu.sync_copy(zbuf.at[pl.ds(0, N - 16 * RPS)],
                            acc.at[pl.ds(16 * RPS, N - 16 * RPS)])

        plsc.subcore_barrier()

        for i in range(NBUF):
            start_idx(i, sets[i])
        for i in range(GP):
            wait_idx(i, sets[i])
            start_gather(sets[i])

        def step(t, i):
            cur = sets[i]
            nxg = sets[(i + GP) % NBUF]
            ibuf, ridx, gbuf, _, _, _ = cur

            @pl.when((t >= 2) & (t - 2 < nch))
            def _free_next_gbuf():
                wait_scatter(nxg)

            @pl.when(t + GP < nch)
            def _prefetch_gather():
                wait_idx(t + GP, nxg)
                start_gather(nxg)

            @pl.when(t < nch)
            def _process():
                wait_gather(cur)

                def edge_body(e, carry):
                    for r in range(H):
                        bw = plsc.bitcast(
                            plsc.load_gather(
                                ibuf,
                                [jnp.full((16,), 2 * H + r, jnp.int32),
                                 jnp.full((16,), e, jnp.int32)]),
                            jnp.float32)
                        for k in range(KV):
                            sl = pl.ds(k * 16, 16)
                            gbuf[r * C + e, sl] = gbuf[r * C + e, sl] * bw
                    return carry

                lax.fori_loop(0, C, edge_body, 0, unroll=4)
                for r in range(H):
                    for k in range(8):
                        sl = pl.ds(k * 16, 16)
                        ridx[r, sl] = ibuf[H + r, sl]
                start_scatter(cur)

            @pl.when(t + NBUF < nch)
            def _prefetch_idx():
                start_idx(t + NBUF, cur)

        def ring_body(u, carry):
            for i in range(NBUF):
                step(NBUF * u + i, i)
            return carry

        lax.fori_loop(0, nchmax // NBUF, ring_body, 0)

        for t in range(nchmax - 2, realmax):
            _st = sets[t % NBUF]

            @pl.when(t < nch)
            def _drain(_st=_st):
                wait_scatter(_st)

        plsc.subcore_barrier()
        pltpu.sync_copy(acc.at[pl.ds(r0, RPS)],
                        out_hbm.at[pl.ds(c * N + r0, RPS)])

        @pl.when(s == 15)
        def _write_tail():
            pltpu.sync_copy(acc.at[pl.ds(16 * RPS, N - 16 * RPS)],
                            out_hbm.at[pl.ds(c * N + 16 * RPS, N - 16 * RPS)])

    return spmm


def _make_dense(Din, Dout, R):
    def body(p_ref, w_ref, b_ref, o_ref):
        x = p_ref[0] + p_ref[1]
        y = jnp.dot(x, w_ref[...], preferred_element_type=jnp.float32,
                    precision=lax.Precision.HIGHEST)
        y = y + b_ref[...]
        nrm = jnp.sqrt(jnp.sum(y * y, axis=1, keepdims=True))
        o_ref[...] = y / jnp.maximum(nrm, 1e-12)

    return pl.pallas_call(
        body,
        grid=(N // R,),
        in_specs=[
            pl.BlockSpec((2, R, Din), lambda i: (0, i, 0)),
            pl.BlockSpec((Din, Dout), lambda i: (0, 0)),
            pl.BlockSpec((1, Dout), lambda i: (0, 0)),
        ],
        out_specs=pl.BlockSpec((R, Dout), lambda i: (i, 0)),
        out_shape=jax.ShapeDtypeStruct((N, Dout), jnp.float32),
    )


_spmm_128 = _make_spmm(128, 3, 1)
_spmm_64 = _make_spmm(64, 4, 1)
_dense_0 = _make_dense(128, 64, 1000)
_dense_1 = _make_dense(64, 128, 1000)


def _pack_idxw(edge_index, edge_weight, H):
    nchd = E // (H * C)
    return jnp.concatenate(
        [edge_index[1].reshape(nchd, H, C),
         edge_index[0].reshape(nchd, H, C),
         lax.bitcast_convert_type(edge_weight, jnp.int32).reshape(
             nchd, H, C)],
        axis=1)


def kernel(fts, edge_index, edge_weight, W_gc_0, b_gc_0, W_gc_1, b_gc_1):
    idxw1 = _pack_idxw(edge_index, edge_weight, 1)
    p0 = _spmm_128(fts, idxw1).reshape(2, N, 128)
    ego = _dense_0(p0, W_gc_0, b_gc_0)
    p1 = _spmm_64(ego, idxw1).reshape(2, N, 64)
    return _dense_1(p1, W_gc_1, b_gc_1)
